# baseline (speedup 1.0000x reference)
import functools

import jax
import jax.numpy as jnp
from jax import lax
from jax.experimental import pallas as pl
from jax.experimental.pallas import tpu as pltpu
from jax.experimental.pallas import tpu_sc as plsc

NUM_NODES = 1000000
EMB = 64
B = 16384

NC = 2
NS = 16
NW = NC * NS
B_PER_W = B // NW
IDX_CHUNK = 128
N_CHUNKS = B_PER_W // IDX_CHUNK
LANES = 16
EMB_VREGS = EMB // LANES


def _sc_kernel(vi_hbm, vj_hbm, a_hbm, c_hbm, out_hbm,
               idx_i, idx_j, rows_a, rows_c, out_v, sem_a, sem_c):
    wid = lax.axis_index("s") * NC + lax.axis_index("c")
    base = wid * B_PER_W

    pltpu.sync_copy(vi_hbm.at[pl.ds(wid * N_CHUNKS, N_CHUNKS)], idx_i)
    pltpu.sync_copy(vj_hbm.at[pl.ds(wid * N_CHUNKS, N_CHUNKS)], idx_j)

    copies = []
    for ch in range(N_CHUNKS):
        dst_a = rows_a.at[pl.ds(ch * IDX_CHUNK, IDX_CHUNK)]
        dst_c = rows_c.at[pl.ds(ch * IDX_CHUNK, IDX_CHUNK)]
        copies.append(pltpu.async_copy(a_hbm.at[idx_i.at[ch]], dst_a, sem_a))
        copies.append(pltpu.async_copy(c_hbm.at[idx_j.at[ch]], dst_c, sem_c))
    for cp in copies:
        cp.wait()

    lane = lax.iota(jnp.int32, LANES)

    def body(g, carry):
        rvec = lane + g * LANES
        acc = jnp.zeros((LANES,), jnp.float32)
        for k in range(EMB):
            kvec = jnp.full((LANES,), k, jnp.int32)
            va = plsc.load_gather(rows_a, [rvec, kvec])
            vc = plsc.load_gather(rows_c, [rvec, kvec])
            acc = acc + va * vc
        out_v[pl.ds(g * LANES, LANES)] = acc
        return carry

    lax.fori_loop(0, B_PER_W // LANES, body, 0)

    pltpu.sync_copy(out_v, out_hbm.at[pl.ds(base, B_PER_W)])


@jax.jit
def kernel(v_i, v_j, first_emb, second_emb, context_emb):
    del first_emb
    vi2 = v_i.reshape(NW * N_CHUNKS, IDX_CHUNK)
    vj2 = v_j.reshape(NW * N_CHUNKS, IDX_CHUNK)
    mesh = plsc.VectorSubcoreMesh(core_axis_name="c", subcore_axis_name="s")
    run = pl.kernel(
        _sc_kernel,
        out_type=jax.ShapeDtypeStruct((B,), jnp.float32),
        mesh=mesh,
        scratch_types=[
            pltpu.VMEM((N_CHUNKS, IDX_CHUNK), jnp.int32),
            pltpu.VMEM((N_CHUNKS, IDX_CHUNK), jnp.int32),
            pltpu.VMEM((B_PER_W, EMB), jnp.float32),
            pltpu.VMEM((B_PER_W, EMB), jnp.float32),
            pltpu.VMEM((B_PER_W,), jnp.float32),
            pltpu.SemaphoreType.DMA,
            pltpu.SemaphoreType.DMA,
        ],
        compiler_params=pltpu.CompilerParams(
            needs_layout_passes=False, use_tc_tiling_on_sc=False),
    )
    return run(vi2, vj2, second_emb, context_emb)
